# baseline (speedup 1.0000x reference)
import functools

import jax
import jax.numpy as jnp
from jax import lax
from jax.experimental import pallas as pl
from jax.experimental.pallas import tpu as pltpu
from jax.experimental.pallas import tpu_sc as plsc

NUM_EMB_DIM = 64
BATCH = 16384
NC = 2
NS = 16
NW = NC * NS
B_W = BATCH // NW
CHUNK = 128
NCHUNK = B_W // CHUNK


def _permute(v, idx16):
    dnums = lax.GatherDimensionNumbers(
        offset_dims=(), collapsed_slice_dims=(0,), start_index_map=(0,))
    return lax.gather(v, idx16[:, None], dnums, slice_sizes=(1,),
                      mode=lax.GatherScatterMode.PROMISE_IN_BOUNDS)


def _mf_kernel(ids1_hbm, ids2_hbm, emb1_hbm, emb2_hbm, out_hbm,
               idx1_v, idx2_v, rows1_v, rows2_v, out_v, sem1, sem2):
    wid = lax.axis_index("s") * NC + lax.axis_index("c")
    base = wid * B_W

    row0 = wid * NCHUNK
    pltpu.sync_copy(ids1_hbm.at[pl.ds(row0, NCHUNK)], idx1_v)
    pltpu.sync_copy(ids2_hbm.at[pl.ds(row0, NCHUNK)], idx2_v)

    copies = []
    for j in range(NCHUNK):
        copies.append(pltpu.async_copy(
            emb1_hbm.at[idx1_v.at[j]], rows1_v.at[pl.ds(j * CHUNK, CHUNK)],
            sem1))
        copies.append(pltpu.async_copy(
            emb2_hbm.at[idx2_v.at[j]], rows2_v.at[pl.ds(j * CHUNK, CHUNK)],
            sem2))
    for c in copies:
        c.wait()

    lane = lax.iota(jnp.int32, 16)
    perms = [jnp.bitwise_xor(lane, s) for s in (8, 4, 2, 1)]

    def body(g, _):
        outvec = jnp.zeros((16,), jnp.float32)
        for r in range(16):
            i = g * 16 + r
            acc = rows1_v[i, pl.ds(0, 16)] * rows2_v[i, pl.ds(0, 16)]
            for c in range(1, NUM_EMB_DIM // 16):
                acc = acc + rows1_v[i, pl.ds(c * 16, 16)] * rows2_v[i, pl.ds(c * 16, 16)]
            for p in perms:
                acc = acc + _permute(acc, p)
            outvec = jnp.where(lane == r, acc, outvec)
        out_v[g] = outvec
        return 0

    lax.fori_loop(0, B_W // 16, body, 0)

    pltpu.sync_copy(out_v, out_hbm.at[pl.ds(wid * (B_W // 16), B_W // 16)])


@jax.jit
def kernel(ids1, ids2, emb1, emb2):
    mesh = plsc.VectorSubcoreMesh(core_axis_name="c", subcore_axis_name="s",
                                  num_cores=NC, num_subcores=NS)
    k = functools.partial(
        pl.kernel,
        out_type=jax.ShapeDtypeStruct((BATCH // 16, 16), jnp.float32),
        mesh=mesh,
        compiler_params=pltpu.CompilerParams(use_tc_tiling_on_sc=False),
        scratch_types=[
            pltpu.VMEM((NCHUNK, CHUNK), jnp.int32),
            pltpu.VMEM((NCHUNK, CHUNK), jnp.int32),
            pltpu.VMEM((B_W, NUM_EMB_DIM), jnp.float32),
            pltpu.VMEM((B_W, NUM_EMB_DIM), jnp.float32),
            pltpu.VMEM((B_W // 16, 16), jnp.float32),
            pltpu.SemaphoreType.DMA,
            pltpu.SemaphoreType.DMA,
        ],
    )(_mf_kernel)
    ids1_2d = ids1.astype(jnp.int32).reshape(BATCH // CHUNK, CHUNK)
    ids2_2d = ids2.astype(jnp.int32).reshape(BATCH // CHUNK, CHUNK)
    out = k(ids1_2d, ids2_2d, emb1, emb2)
    return out.reshape(BATCH, 1)


# speedup vs baseline: 1.5644x; 1.5644x over previous
import functools

import jax
import jax.numpy as jnp
from jax import lax
from jax.experimental import pallas as pl
from jax.experimental.pallas import tpu as pltpu
from jax.experimental.pallas import tpu_sc as plsc

EMB_D = 64
BATCH = 16384
NC = 2
NS = 16
NW = NC * NS
B_W = BATCH // NW
CH = 128
NCHUNK = B_W // CH


def _permute(v, idx16):
    dnums = lax.GatherDimensionNumbers(
        offset_dims=(), collapsed_slice_dims=(0,), start_index_map=(0,))
    return lax.gather(v, idx16[:, None], dnums, slice_sizes=(1,),
                      mode=lax.GatherScatterMode.PROMISE_IN_BOUNDS)


def _mf_kernel(ids1_hbm, ids2_hbm, emb1_hbm, emb2_hbm, out_hbm,
               idx1_v, idx2_v, rows1_v, rows2_v, out_v, sem1, sem2):
    wid = lax.axis_index("s") * NC + lax.axis_index("c")

    pltpu.sync_copy(ids1_hbm.at[wid], idx1_v)
    pltpu.sync_copy(ids2_hbm.at[wid], idx2_v)

    lane = lax.iota(jnp.int32, 16)
    perms = [jnp.bitwise_xor(lane, s) for s in (8, 4, 2, 1)]

    def chunk_step(ci, _):
        base = ci * CH

        def issue(g, _):
            vec1 = idx1_v[pl.ds(base + g * 16, 16)]
            vec2 = idx2_v[pl.ds(base + g * 16, 16)]
            for r in range(16):
                j = g * 16 + r
                pltpu.async_copy(emb1_hbm.at[vec1[r]], rows1_v.at[j], sem1)
                pltpu.async_copy(emb2_hbm.at[vec2[r]], rows2_v.at[j], sem2)
            return 0

        lax.fori_loop(0, CH // 16, issue, 0)

        def drain(j, _):
            pltpu.make_async_copy(emb1_hbm.at[0], rows1_v.at[j], sem1).wait()
            pltpu.make_async_copy(emb2_hbm.at[0], rows2_v.at[j], sem2).wait()
            return 0

        lax.fori_loop(0, CH, drain, 0)

        def body(g, _):
            outvec = jnp.zeros((16,), jnp.float32)
            for r in range(16):
                j = g * 16 + r
                acc = rows1_v[j, pl.ds(0, 16)] * rows2_v[j, pl.ds(0, 16)]
                for c in range(1, EMB_D // 16):
                    acc = acc + (rows1_v[j, pl.ds(c * 16, 16)]
                                 * rows2_v[j, pl.ds(c * 16, 16)])
                for p in perms:
                    acc = acc + _permute(acc, p)
                outvec = jnp.where(lane == r, acc, outvec)
            out_v[pl.ds(base + g * 16, 16)] = outvec
            return 0

        lax.fori_loop(0, CH // 16, body, 0)
        return 0

    lax.fori_loop(0, NCHUNK, chunk_step, 0)

    pltpu.sync_copy(out_v, out_hbm.at[wid])


@jax.jit
def kernel(ids1, ids2, emb1, emb2):
    mesh = plsc.VectorSubcoreMesh(core_axis_name="c", subcore_axis_name="s",
                                  num_cores=NC, num_subcores=NS)
    k = functools.partial(
        pl.kernel,
        out_type=jax.ShapeDtypeStruct((NW, B_W), jnp.float32),
        mesh=mesh,
        scratch_types=[
            pltpu.VMEM((B_W,), jnp.int32),
            pltpu.VMEM((B_W,), jnp.int32),
            pltpu.VMEM((CH, EMB_D), jnp.float32),
            pltpu.VMEM((CH, EMB_D), jnp.float32),
            pltpu.VMEM((B_W,), jnp.float32),
            pltpu.SemaphoreType.DMA,
            pltpu.SemaphoreType.DMA,
        ],
    )(_mf_kernel)
    ids1_2d = ids1.astype(jnp.int32).reshape(NW, B_W)
    ids2_2d = ids2.astype(jnp.int32).reshape(NW, B_W)
    out = k(ids1_2d, ids2_2d, emb1, emb2)
    return out.reshape(BATCH, 1)
